# baseline (speedup 1.0000x reference)
import jax
import jax.numpy as jnp
from jax.experimental import pallas as pl

_BB = 256


def _body(x_ref, pos_ref, o_ref):
    n, l = x_ref.shape
    d = pos_ref.shape[0]
    y = x_ref[...][:, None, :] + pos_ref[...][None, :, :]
    o_ref[...] = y.reshape(n * d, l)


def kernel(x, pos_table):
    B, L = x.shape
    D = pos_table.shape[-1]
    posT = pos_table.T
    y = pl.pallas_call(
        _body,
        grid=(B // _BB,),
        in_specs=[
            pl.BlockSpec((_BB, L), lambda i: (i, 0)),
            pl.BlockSpec((D, L), lambda i: (0, 0)),
        ],
        out_specs=pl.BlockSpec((_BB * D, L), lambda i: (i, 0)),
        out_shape=jax.ShapeDtypeStruct((B * D, L), x.dtype),
    )(x, posT)
    return y


# speedup vs baseline: 4.4212x; 4.4212x over previous
import jax
import jax.numpy as jnp
from jax.experimental import pallas as pl

_RBX = 8


def _body(xt_ref, pos_ref, o_ref):
    nx, nb = xt_ref.shape
    nr = o_ref.shape[0]
    d = nr // nx
    xt = xt_ref[...]
    y = jnp.broadcast_to(xt[:, None, :], (nx, d, nb)).reshape(nr, nb)
    o_ref[...] = y + jnp.broadcast_to(pos_ref[...], (nr, nb))


def kernel(x, pos_table):
    B, L = x.shape
    D = pos_table.shape[-1]
    xt = x.T
    pos_col = pos_table.reshape(L * D, 1)
    rb = _RBX * D
    y = pl.pallas_call(
        _body,
        grid=(L // _RBX,),
        in_specs=[
            pl.BlockSpec((_RBX, B), lambda i: (i, 0)),
            pl.BlockSpec((rb, 1), lambda i: (i, 0)),
        ],
        out_specs=pl.BlockSpec((rb, B), lambda i: (i, 0)),
        out_shape=jax.ShapeDtypeStruct((L * D, B), x.dtype),
    )(xt, pos_col)
    return y.reshape(L, D, B).transpose(2, 0, 1)
